# baseline (speedup 1.0000x reference)
import functools
import jax
import jax.numpy as jnp
from jax import lax
from jax.experimental import pallas as pl
from jax.experimental.pallas import tpu as pltpu
from jax.experimental.pallas import tpu_sc as plsc

N = 10000
E = 320000
IN = 128
H = 4
C = 32
HID = H * C
NEG = 0.2

NPAD = 10240
NC = 2
NS = 16
NW = NC * NS
B = 80
EPW = 10080
EPAD = EPW * NW
NCHUNK = EPW // B
NPAIR = NCHUNK // 2
RPT = NPAD // NS
DL = 16
BLK = 10240
NBLK = NPAD // BLK


def _proj_body(x_ref, w_ref, aw1_ref, aw2_ref, proj_ref, at1_ref, at2_ref):
    p = jnp.dot(x_ref[...], w_ref[...], preferred_element_type=jnp.float32)
    proj_ref[...] = p
    at1_ref[...] = jnp.dot(p, aw1_ref[...], preferred_element_type=jnp.float32)
    at2_ref[...] = jnp.dot(p, aw2_ref[...], preferred_element_type=jnp.float32)


def _final_body(acc_ref, den_ref, proj_ref, at1_ref, at2_ref, er4_ref,
                er16_ref, b_ref, o_ref):
    l = at1_ref[:, :H] + at2_ref[:, :H]
    l = jnp.where(l >= 0.0, l, l * NEG)
    ews = jnp.exp(l)
    ews_e = jnp.dot(ews, er4_ref[...], preferred_element_type=jnp.float32)
    den = den_ref[0] + den_ref[1]
    den_e = jnp.dot(den, er16_ref[...], preferred_element_type=jnp.float32)
    den_e = den_e + ews_e
    acc = acc_ref[0] + acc_ref[1] + ews_e * proj_ref[...]
    o_ref[...] = acc / den_e + b_ref[...]


_sc_mesh = plsc.VectorSubcoreMesh(core_axis_name="c", subcore_axis_name="s")


@functools.partial(
    pl.kernel,
    out_type=(
        jax.ShapeDtypeStruct((NC, NPAD, HID), jnp.float32),
        jax.ShapeDtypeStruct((NC, NPAD, DL), jnp.float32),
    ),
    mesh=_sc_mesh,
    compiler_params=pltpu.CompilerParams(use_tc_tiling_on_sc=False,
                                         needs_layout_passes=False),
    scratch_types=[
        [pltpu.VMEM((B, HID), jnp.float32) for _ in range(3)],
        [pltpu.VMEM((B, DL), jnp.float32) for _ in range(2)],
        [pltpu.VMEM((B, DL), jnp.float32) for _ in range(2)],
        [pltpu.VMEM((B,), jnp.int32) for _ in range(2)],
        [pltpu.VMEM((B,), jnp.int32) for _ in range(2)],
        [pltpu.VMEM((B,), jnp.int32) for _ in range(2)],
        [pltpu.VMEM((B, DL), jnp.float32) for _ in range(2)],
        pltpu.VMEM_SHARED((NPAD, HID), jnp.float32),
        pltpu.VMEM_SHARED((NPAD, DL), jnp.float32),
        [pltpu.SemaphoreType.DMA for _ in range(2)],
        [pltpu.SemaphoreType.DMA for _ in range(2)],
        [pltpu.SemaphoreType.DMA for _ in range(2)],
        [pltpu.SemaphoreType.DMA for _ in range(2)],
    ],
)
def _sc_gat(proj_hbm, at1_hbm, at2_hbm, row_hbm, col_hbm, acc_hbm, den_hbm,
            rows, a1, a2, ridx, cidx, csc, ew, acc_sh, den_sh,
            sg, sa, si, ss):
    c = lax.axis_index("c")
    s = lax.axis_index("s")
    wid = c * NS + s
    ebase = wid * EPW

    zf = jnp.zeros((16,), jnp.float32)

    def _zrow(i, carry):
        for v in range(HID // 16):
            rows[0][i, pl.ds(v * 16, 16)] = zf
        ew[0][i, pl.ds(0, 16)] = zf
        return carry

    lax.fori_loop(0, B, _zrow, 0)

    rbase = s * RPT
    for k in range(RPT // B):
        pltpu.async_copy(rows[0], acc_sh.at[pl.ds(rbase + k * B, B)], sg[0])
        pltpu.async_copy(ew[0], den_sh.at[pl.ds(rbase + k * B, B)], sg[0])
    for k in range(RPT // B):
        pltpu.make_async_copy(rows[0], acc_sh.at[pl.ds(rbase + k * B, B)], sg[0]).wait()
        pltpu.make_async_copy(ew[0], den_sh.at[pl.ds(rbase + k * B, B)], sg[0]).wait()
    plsc.subcore_barrier()

    lanemask = (lax.iota(jnp.int32, 16) < H).astype(jnp.float32)

    def issue_idx(t, m):
        base = ebase + t * B
        pltpu.async_copy(row_hbm.at[pl.ds(base, B)], ridx[m], si[m])
        pltpu.async_copy(col_hbm.at[pl.ds(base, B)], cidx[m], si[m])

    def wait_idx(m):
        pltpu.make_async_copy(row_hbm.at[pl.ds(0, B)], ridx[m], si[m]).wait()
        pltpu.make_async_copy(col_hbm.at[pl.ds(0, B)], cidx[m], si[m]).wait()

    def issue_gathers(m3, m2):
        pltpu.async_copy(proj_hbm.at[ridx[m2]], rows[m3], sg[m2])
        pltpu.async_copy(at1_hbm.at[ridx[m2]], a1[m2], sa[m2])
        pltpu.async_copy(at2_hbm.at[cidx[m2]], a2[m2], sa[m2])

    def wait_alpha(m2):
        pltpu.make_async_copy(at1_hbm.at[pl.ds(0, B)], a1[m2], sa[m2]).wait()
        pltpu.make_async_copy(at2_hbm.at[pl.ds(0, B)], a2[m2], sa[m2]).wait()

    def wait_rows(m3, m2):
        pltpu.make_async_copy(proj_hbm.at[pl.ds(0, B)], rows[m3], sg[m2]).wait()

    def compute_ew(m2):
        @plsc.parallel_loop(0, B // 16, 1, unroll=5)
        def _grp(g):
            rv = ridx[m2][pl.ds(g * 16, 16)]
            cv = cidx[m2][pl.ds(g * 16, 16)]
            validf = jnp.where(rv != cv, 1.0, 0.0)
            for j in range(16):
                e = g * 16 + j
                l = a1[m2][e, pl.ds(0, 16)] + a2[m2][e, pl.ds(0, 16)]
                l = jnp.where(l >= 0.0, l, l * NEG)
                w = jnp.exp(l) * validf[j] * lanemask
                ew[m2][e, pl.ds(0, 16)] = w

    def copy_csc(m2):
        for k in range(B // 16):
            csc[m2][pl.ds(k * 16, 16)] = cidx[m2][pl.ds(k * 16, 16)]

    def scale(m3, m2):
        @plsc.parallel_loop(0, B, 1, unroll=8)
        def _sc(e):
            wv = ew[m2][e, pl.ds(0, 16)]
            for h in range(H):
                wsc = wv[h]
                for v in range(2 * h, 2 * h + 2):
                    rows[m3][e, pl.ds(v * 16, 16)] = (
                        rows[m3][e, pl.ds(v * 16, 16)] * wsc)

    def issue_scatter(m3, m2):
        pltpu.async_copy(rows[m3], acc_sh.at[csc[m2]], ss[m2], add=True)
        pltpu.async_copy(ew[m2], den_sh.at[csc[m2]], ss[m2], add=True)

    def wait_scatter(m3, m2):
        pltpu.make_async_copy(rows[m3], acc_sh.at[csc[m2]], ss[m2]).wait()
        pltpu.make_async_copy(ew[m2], den_sh.at[csc[m2]], ss[m2]).wait()

    NI = NCHUNK // 6

    issue_idx(0, 0)
    wait_idx(0)
    issue_gathers(0, 0)
    issue_idx(1, 1)

    def group(i, carry):
        for k in range(6):
            t = 6 * i + k
            p3, p2 = k % 3, k % 2
            q3, q2 = (k + 1) % 3, (k + 1) % 2

            if k < 2:
                @pl.when(i > 0)
                def _(k=k, q3=q3, p2=p2):
                    wait_scatter(q3, p2)
            else:
                wait_scatter(q3, p2)

            if k == 5:
                @pl.when(i < NI - 1)
                def _(q3=q3, q2=q2):
                    wait_idx(q2)
                    issue_gathers(q3, q2)
            else:
                wait_idx(q2)
                issue_gathers(q3, q2)

            wait_alpha(p2)
            compute_ew(p2)

            wait_rows(p3, p2)
            copy_csc(p2)
            scale(p3, p2)

            if k >= 4:
                @pl.when(i < NI - 1)
                def _(t=t, p2=p2):
                    issue_idx(t + 2, p2)
            else:
                issue_idx(t + 2, p2)

            issue_scatter(p3, p2)
        return carry

    lax.fori_loop(0, NI, group, 0)
    wait_scatter(1, 0)
    wait_scatter(2, 1)

    plsc.subcore_barrier()
    pltpu.async_copy(acc_sh.at[pl.ds(rbase, RPT)], acc_hbm.at[c, pl.ds(rbase, RPT)], sg[0])
    pltpu.async_copy(den_sh.at[pl.ds(rbase, RPT)], den_hbm.at[c, pl.ds(rbase, RPT)], sg[1])
    pltpu.make_async_copy(acc_sh.at[pl.ds(rbase, RPT)], acc_hbm.at[c, pl.ds(rbase, RPT)], sg[0]).wait()
    pltpu.make_async_copy(den_sh.at[pl.ds(rbase, RPT)], den_hbm.at[c, pl.ds(rbase, RPT)], sg[1]).wait()


@jax.jit
def _run(x, edge_index, W, att_src, att_dst, b):
    xp = jnp.zeros((NPAD, IN), jnp.float32).at[:N].set(x)
    eye4 = jnp.eye(H, dtype=jnp.float32)
    aw_src = (att_src[:, :, None] * eye4[:, None, :]).reshape(HID, H)
    aw_dst = (att_dst[:, :, None] * eye4[:, None, :]).reshape(HID, H)
    zpad = jnp.zeros((HID, DL - H), jnp.float32)
    aw1 = jnp.concatenate([aw_src, zpad], axis=1)
    aw2 = jnp.concatenate([aw_dst, zpad], axis=1)

    proj, at1, at2 = pl.pallas_call(
        _proj_body,
        grid=(NBLK,),
        in_specs=[
            pl.BlockSpec((BLK, IN), lambda i: (i, 0)),
            pl.BlockSpec((IN, HID), lambda i: (0, 0)),
            pl.BlockSpec((HID, DL), lambda i: (0, 0)),
            pl.BlockSpec((HID, DL), lambda i: (0, 0)),
        ],
        out_specs=[
            pl.BlockSpec((BLK, HID), lambda i: (i, 0)),
            pl.BlockSpec((BLK, DL), lambda i: (i, 0)),
            pl.BlockSpec((BLK, DL), lambda i: (i, 0)),
        ],
        out_shape=[
            jax.ShapeDtypeStruct((NPAD, HID), jnp.float32),
            jax.ShapeDtypeStruct((NPAD, DL), jnp.float32),
            jax.ShapeDtypeStruct((NPAD, DL), jnp.float32),
        ],
    )(xp, W, aw1, aw2)

    pad_ids = N + (jnp.arange(EPAD - E, dtype=jnp.int32) % (NPAD - N))
    row = jnp.concatenate([edge_index[0], pad_ids])
    col = jnp.concatenate([edge_index[1], pad_ids])
    acc, den = _sc_gat(proj, at1, at2, row, col)

    er4 = jnp.repeat(jnp.eye(H, dtype=jnp.float32), C, axis=1)
    er16 = jnp.zeros((DL, HID), jnp.float32).at[:H].set(er4)
    b2 = b.reshape(1, HID)

    out = pl.pallas_call(
        _final_body,
        grid=(NBLK,),
        in_specs=[
            pl.BlockSpec((NC, BLK, HID), lambda i: (0, i, 0)),
            pl.BlockSpec((NC, BLK, DL), lambda i: (0, i, 0)),
            pl.BlockSpec((BLK, HID), lambda i: (i, 0)),
            pl.BlockSpec((BLK, DL), lambda i: (i, 0)),
            pl.BlockSpec((BLK, DL), lambda i: (i, 0)),
            pl.BlockSpec((H, HID), lambda i: (0, 0)),
            pl.BlockSpec((DL, HID), lambda i: (0, 0)),
            pl.BlockSpec((1, HID), lambda i: (0, 0)),
        ],
        out_specs=pl.BlockSpec((BLK, HID), lambda i: (i, 0)),
        out_shape=jax.ShapeDtypeStruct((NPAD, HID), jnp.float32),
    )(acc, den, proj, at1, at2, er4, er16, b2)

    return out[:N]


def kernel(x, edge_index, W, att_src, att_dst, b):
    return _run(x, edge_index, W, att_src, att_dst, b)


# speedup vs baseline: 1.1540x; 1.1540x over previous
import functools
import jax
import jax.numpy as jnp
from jax import lax
from jax.experimental import pallas as pl
from jax.experimental.pallas import tpu as pltpu
from jax.experimental.pallas import tpu_sc as plsc

N = 10000
E = 320000
IN = 128
H = 4
C = 32
HID = H * C
NEG = 0.2

NPAD = 10240
NC = 2
NS = 16
NW = NC * NS
B = 80
EPW = 10080
EPAD = EPW * NW
NCHUNK = EPW // B
NPAIR = NCHUNK // 2
RPT = NPAD // NS
DL = 16
BLK = 10240
NBLK = NPAD // BLK


def _proj_body(x_ref, w_ref, aw1_ref, aw2_ref, proj_ref, at1_ref, at2_ref):
    p = jnp.dot(x_ref[...], w_ref[...], preferred_element_type=jnp.float32)
    proj_ref[...] = p
    at1_ref[...] = jnp.dot(p, aw1_ref[...], preferred_element_type=jnp.float32)
    at2_ref[...] = jnp.dot(p, aw2_ref[...], preferred_element_type=jnp.float32)


def _final_body(acc_ref, den_ref, proj_ref, at1_ref, at2_ref, er4_ref,
                er16_ref, b_ref, o_ref):
    l = at1_ref[:, :H] + at2_ref[:, :H]
    l = jnp.where(l >= 0.0, l, l * NEG)
    ews = jnp.exp(l)
    ews_e = jnp.dot(ews, er4_ref[...], preferred_element_type=jnp.float32)
    den = den_ref[0] + den_ref[1]
    den_e = jnp.dot(den, er16_ref[...], preferred_element_type=jnp.float32)
    den_e = den_e + ews_e
    acc = acc_ref[0] + acc_ref[1] + ews_e * proj_ref[...]
    o_ref[...] = acc / den_e + b_ref[...]


_sc_mesh = plsc.VectorSubcoreMesh(core_axis_name="c", subcore_axis_name="s")


@functools.partial(
    pl.kernel,
    out_type=(
        jax.ShapeDtypeStruct((NC, NPAD, HID), jnp.float32),
        jax.ShapeDtypeStruct((NC, NPAD, DL), jnp.float32),
    ),
    mesh=_sc_mesh,
    compiler_params=pltpu.CompilerParams(use_tc_tiling_on_sc=False,
                                         needs_layout_passes=False),
    scratch_types=[
        [pltpu.VMEM((B, HID), jnp.float32) for _ in range(3)],
        [pltpu.VMEM((B, DL), jnp.float32) for _ in range(2)],
        [pltpu.VMEM((B, DL), jnp.float32) for _ in range(2)],
        [pltpu.VMEM((B,), jnp.int32) for _ in range(2)],
        [pltpu.VMEM((B,), jnp.int32) for _ in range(2)],
        [pltpu.VMEM((B,), jnp.int32) for _ in range(2)],
        [pltpu.VMEM((B, DL), jnp.float32) for _ in range(2)],
        pltpu.VMEM_SHARED((NPAD, HID), jnp.float32),
        pltpu.VMEM_SHARED((NPAD, DL), jnp.float32),
        [pltpu.SemaphoreType.DMA for _ in range(2)],
        [pltpu.SemaphoreType.DMA for _ in range(2)],
        [pltpu.SemaphoreType.DMA for _ in range(2)],
        [pltpu.SemaphoreType.DMA for _ in range(2)],
    ],
)
def _sc_gat(proj_hbm, at1_hbm, at2_hbm, row_hbm, col_hbm, acc_hbm, den_hbm,
            rows, a1, a2, ridx, cidx, csc, ew, acc_sh, den_sh,
            sg, sa, si, ss):
    c = lax.axis_index("c")
    s = lax.axis_index("s")
    wid = c * NS + s
    ebase = wid * EPW

    zf = jnp.zeros((16,), jnp.float32)

    def _zrow(i, carry):
        for v in range(HID // 16):
            rows[0][i, pl.ds(v * 16, 16)] = zf
        ew[0][i, pl.ds(0, 16)] = zf
        return carry

    lax.fori_loop(0, B, _zrow, 0)

    rbase = s * RPT
    for k in range(RPT // B):
        pltpu.async_copy(rows[0], acc_sh.at[pl.ds(rbase + k * B, B)], sg[0])
        pltpu.async_copy(ew[0], den_sh.at[pl.ds(rbase + k * B, B)], sg[0])
    for k in range(RPT // B):
        pltpu.make_async_copy(rows[0], acc_sh.at[pl.ds(rbase + k * B, B)], sg[0]).wait()
        pltpu.make_async_copy(ew[0], den_sh.at[pl.ds(rbase + k * B, B)], sg[0]).wait()
    plsc.subcore_barrier()

    def issue_idx(t, m):
        base = ebase + t * B
        pltpu.async_copy(row_hbm.at[pl.ds(base, B)], ridx[m], si[m])
        pltpu.async_copy(col_hbm.at[pl.ds(base, B)], cidx[m], si[m])

    def wait_idx(m):
        pltpu.make_async_copy(row_hbm.at[pl.ds(0, B)], ridx[m], si[m]).wait()
        pltpu.make_async_copy(col_hbm.at[pl.ds(0, B)], cidx[m], si[m]).wait()

    def issue_gathers(m3, m2):
        pltpu.async_copy(proj_hbm.at[ridx[m2]], rows[m3], sg[m2])
        pltpu.async_copy(at1_hbm.at[ridx[m2]], a1[m2], sa[m2])
        pltpu.async_copy(at2_hbm.at[cidx[m2]], a2[m2], sa[m2])

    def wait_alpha(m2):
        pltpu.make_async_copy(at1_hbm.at[pl.ds(0, B)], a1[m2], sa[m2]).wait()
        pltpu.make_async_copy(at2_hbm.at[pl.ds(0, B)], a2[m2], sa[m2]).wait()

    def wait_rows(m3, m2):
        pltpu.make_async_copy(proj_hbm.at[pl.ds(0, B)], rows[m3], sg[m2]).wait()

    def compute_ew(m2):
        @plsc.parallel_loop(0, B // 16, 1, unroll=2)
        def _grp(g):
            rv = ridx[m2][pl.ds(g * 16, 16)]
            cv = cidx[m2][pl.ds(g * 16, 16)]
            validf = jnp.where(rv != cv, 1.0, 0.0)
            for j in range(16):
                e = g * 16 + j
                l = a1[m2][e, pl.ds(0, 16)] + a2[m2][e, pl.ds(0, 16)]
                l = jnp.maximum(l, l * NEG)
                w = jnp.exp(l) * validf[j]
                ew[m2][e, pl.ds(0, 16)] = w

    def copy_csc(m2):
        for k in range(B // 16):
            csc[m2][pl.ds(k * 16, 16)] = cidx[m2][pl.ds(k * 16, 16)]

    def scale(m3, m2):
        @plsc.parallel_loop(0, B, 1, unroll=4)
        def _sc(e):
            wv = ew[m2][e, pl.ds(0, 16)]
            for h in range(H):
                wsc = wv[h]
                for v in range(2 * h, 2 * h + 2):
                    rows[m3][e, pl.ds(v * 16, 16)] = (
                        rows[m3][e, pl.ds(v * 16, 16)] * wsc)

    def issue_scatter(m3, m2):
        pltpu.async_copy(rows[m3], acc_sh.at[csc[m2]], ss[m2], add=True)
        pltpu.async_copy(ew[m2], den_sh.at[csc[m2]], ss[m2], add=True)

    def wait_scatter(m3, m2):
        pltpu.make_async_copy(rows[m3], acc_sh.at[csc[m2]], ss[m2]).wait()
        pltpu.make_async_copy(ew[m2], den_sh.at[csc[m2]], ss[m2]).wait()

    NI = NCHUNK // 6

    issue_idx(0, 0)
    wait_idx(0)
    issue_gathers(0, 0)
    issue_idx(1, 1)

    def group(i, carry):
        for k in range(6):
            t = 6 * i + k
            p3, p2 = k % 3, k % 2
            q3, q2 = (k + 1) % 3, (k + 1) % 2

            if k < 2:
                @pl.when(i > 0)
                def _(k=k, q3=q3, p2=p2):
                    wait_scatter(q3, p2)
            else:
                wait_scatter(q3, p2)

            if k == 5:
                @pl.when(i < NI - 1)
                def _(q3=q3, q2=q2):
                    wait_idx(q2)
                    issue_gathers(q3, q2)
            else:
                wait_idx(q2)
                issue_gathers(q3, q2)

            wait_alpha(p2)
            compute_ew(p2)

            wait_rows(p3, p2)
            copy_csc(p2)
            scale(p3, p2)

            if k >= 4:
                @pl.when(i < NI - 1)
                def _(t=t, p2=p2):
                    issue_idx(t + 2, p2)
            else:
                issue_idx(t + 2, p2)

            issue_scatter(p3, p2)
        return carry

    lax.fori_loop(0, NI, group, 0)
    wait_scatter(1, 0)
    wait_scatter(2, 1)

    plsc.subcore_barrier()
    pltpu.async_copy(acc_sh.at[pl.ds(rbase, RPT)], acc_hbm.at[c, pl.ds(rbase, RPT)], sg[0])
    pltpu.async_copy(den_sh.at[pl.ds(rbase, RPT)], den_hbm.at[c, pl.ds(rbase, RPT)], sg[1])
    pltpu.make_async_copy(acc_sh.at[pl.ds(rbase, RPT)], acc_hbm.at[c, pl.ds(rbase, RPT)], sg[0]).wait()
    pltpu.make_async_copy(den_sh.at[pl.ds(rbase, RPT)], den_hbm.at[c, pl.ds(rbase, RPT)], sg[1]).wait()


@jax.jit
def _run(x, edge_index, W, att_src, att_dst, b):
    xp = jnp.zeros((NPAD, IN), jnp.float32).at[:N].set(x)
    eye4 = jnp.eye(H, dtype=jnp.float32)
    aw_src = (att_src[:, :, None] * eye4[:, None, :]).reshape(HID, H)
    aw_dst = (att_dst[:, :, None] * eye4[:, None, :]).reshape(HID, H)
    zpad = jnp.zeros((HID, DL - H), jnp.float32)
    aw1 = jnp.concatenate([aw_src, zpad], axis=1)
    aw2 = jnp.concatenate([aw_dst, zpad], axis=1)

    proj, at1, at2 = pl.pallas_call(
        _proj_body,
        grid=(NBLK,),
        in_specs=[
            pl.BlockSpec((BLK, IN), lambda i: (i, 0)),
            pl.BlockSpec((IN, HID), lambda i: (0, 0)),
            pl.BlockSpec((HID, DL), lambda i: (0, 0)),
            pl.BlockSpec((HID, DL), lambda i: (0, 0)),
        ],
        out_specs=[
            pl.BlockSpec((BLK, HID), lambda i: (i, 0)),
            pl.BlockSpec((BLK, DL), lambda i: (i, 0)),
            pl.BlockSpec((BLK, DL), lambda i: (i, 0)),
        ],
        out_shape=[
            jax.ShapeDtypeStruct((NPAD, HID), jnp.float32),
            jax.ShapeDtypeStruct((NPAD, DL), jnp.float32),
            jax.ShapeDtypeStruct((NPAD, DL), jnp.float32),
        ],
    )(xp, W, aw1, aw2)

    pad_ids = N + (jnp.arange(EPAD - E, dtype=jnp.int32) % (NPAD - N))
    row = jnp.concatenate([edge_index[0], pad_ids])
    col = jnp.concatenate([edge_index[1], pad_ids])
    acc, den = _sc_gat(proj, at1, at2, row, col)

    er4 = jnp.repeat(jnp.eye(H, dtype=jnp.float32), C, axis=1)
    er16 = jnp.zeros((DL, HID), jnp.float32).at[:H].set(er4)
    b2 = b.reshape(1, HID)

    out = pl.pallas_call(
        _final_body,
        grid=(NBLK,),
        in_specs=[
            pl.BlockSpec((NC, BLK, HID), lambda i: (0, i, 0)),
            pl.BlockSpec((NC, BLK, DL), lambda i: (0, i, 0)),
            pl.BlockSpec((BLK, HID), lambda i: (i, 0)),
            pl.BlockSpec((BLK, DL), lambda i: (i, 0)),
            pl.BlockSpec((BLK, DL), lambda i: (i, 0)),
            pl.BlockSpec((H, HID), lambda i: (0, 0)),
            pl.BlockSpec((DL, HID), lambda i: (0, 0)),
            pl.BlockSpec((1, HID), lambda i: (0, 0)),
        ],
        out_specs=pl.BlockSpec((BLK, HID), lambda i: (i, 0)),
        out_shape=jax.ShapeDtypeStruct((NPAD, HID), jnp.float32),
    )(acc, den, proj, at1, at2, er4, er16, b2)

    return out[:N]


def kernel(x, edge_index, W, att_src, att_dst, b):
    return _run(x, edge_index, W, att_src, att_dst, b)


# speedup vs baseline: 1.1793x; 1.0219x over previous
import functools
import jax
import jax.numpy as jnp
from jax import lax
from jax.experimental import pallas as pl
from jax.experimental.pallas import tpu as pltpu
from jax.experimental.pallas import tpu_sc as plsc

N = 10000
E = 320000
IN = 128
H = 4
C = 32
HID = H * C
NEG = 0.2

NPAD = 10240
NC = 2
NS = 16
NW = NC * NS
B = 80
EPW = 10080
EPAD = EPW * NW
NCHUNK = EPW // B
NPAIR = NCHUNK // 2
RPT = NPAD // NS
DL = 16
BLK = 10240
NBLK = NPAD // BLK


def _proj_body(x_ref, w_ref, aw1_ref, aw2_ref, proj_ref, at1_ref, at2_ref):
    p = jnp.dot(x_ref[...], w_ref[...], preferred_element_type=jnp.float32)
    proj_ref[pl.ds(0, N)] = p
    proj_ref[pl.ds(N, NPAD - N)] = jnp.zeros((NPAD - N, HID), jnp.float32)
    at1_ref[pl.ds(0, N)] = jnp.dot(p, aw1_ref[...],
                                   preferred_element_type=jnp.float32)
    at1_ref[pl.ds(N, NPAD - N)] = jnp.zeros((NPAD - N, DL), jnp.float32)
    at2_ref[pl.ds(0, N)] = jnp.dot(p, aw2_ref[...],
                                   preferred_element_type=jnp.float32)
    at2_ref[pl.ds(N, NPAD - N)] = jnp.zeros((NPAD - N, DL), jnp.float32)


def _final_body(acc_ref, den_ref, proj_ref, at1_ref, at2_ref, er4_ref,
                er16_ref, b_ref, o_ref):
    l = at1_ref[pl.ds(0, N), :H] + at2_ref[pl.ds(0, N), :H]
    l = jnp.maximum(l, l * NEG)
    ews = jnp.exp(l)
    ews_e = jnp.dot(ews, er4_ref[...], preferred_element_type=jnp.float32)
    den = den_ref[0, pl.ds(0, N)] + den_ref[1, pl.ds(0, N)]
    den_e = jnp.dot(den, er16_ref[...], preferred_element_type=jnp.float32)
    den_e = den_e + ews_e
    acc = (acc_ref[0, pl.ds(0, N)] + acc_ref[1, pl.ds(0, N)]
           + ews_e * proj_ref[pl.ds(0, N)])
    o_ref[...] = acc / den_e + b_ref[...]


_sc_mesh = plsc.VectorSubcoreMesh(core_axis_name="c", subcore_axis_name="s")


@functools.partial(
    pl.kernel,
    out_type=(
        jax.ShapeDtypeStruct((NC, NPAD, HID), jnp.float32),
        jax.ShapeDtypeStruct((NC, NPAD, DL), jnp.float32),
    ),
    mesh=_sc_mesh,
    compiler_params=pltpu.CompilerParams(use_tc_tiling_on_sc=False,
                                         needs_layout_passes=False),
    scratch_types=[
        [pltpu.VMEM((B, HID), jnp.float32) for _ in range(3)],
        [pltpu.VMEM((B, DL), jnp.float32) for _ in range(2)],
        [pltpu.VMEM((B, DL), jnp.float32) for _ in range(2)],
        [pltpu.VMEM((B,), jnp.int32) for _ in range(2)],
        [pltpu.VMEM((B,), jnp.int32) for _ in range(2)],
        [pltpu.VMEM((B,), jnp.int32) for _ in range(2)],
        [pltpu.VMEM((B, DL), jnp.float32) for _ in range(2)],
        pltpu.VMEM_SHARED((NPAD, HID), jnp.float32),
        pltpu.VMEM_SHARED((NPAD, DL), jnp.float32),
        [pltpu.SemaphoreType.DMA for _ in range(2)],
        [pltpu.SemaphoreType.DMA for _ in range(2)],
        [pltpu.SemaphoreType.DMA for _ in range(2)],
        [pltpu.SemaphoreType.DMA for _ in range(2)],
    ],
)
def _sc_gat(proj_hbm, at1_hbm, at2_hbm, row_hbm, col_hbm, acc_hbm, den_hbm,
            rows, a1, a2, ridx, cidx, csc, ew, acc_sh, den_sh,
            sg, sa, si, ss):
    c = lax.axis_index("c")
    s = lax.axis_index("s")
    wid = c * NS + s
    ebase = wid * EPW

    zf = jnp.zeros((16,), jnp.float32)

    def _zrow(i, carry):
        for v in range(HID // 16):
            rows[0][i, pl.ds(v * 16, 16)] = zf
        ew[0][i, pl.ds(0, 16)] = zf
        return carry

    lax.fori_loop(0, B, _zrow, 0)

    rbase = s * RPT
    for k in range(RPT // B):
        pltpu.async_copy(rows[0], acc_sh.at[pl.ds(rbase + k * B, B)], sg[0])
        pltpu.async_copy(ew[0], den_sh.at[pl.ds(rbase + k * B, B)], sg[0])
    for k in range(RPT // B):
        pltpu.make_async_copy(rows[0], acc_sh.at[pl.ds(rbase + k * B, B)], sg[0]).wait()
        pltpu.make_async_copy(ew[0], den_sh.at[pl.ds(rbase + k * B, B)], sg[0]).wait()
    plsc.subcore_barrier()

    def issue_idx(t, m):
        base = ebase + t * B
        pltpu.async_copy(row_hbm.at[pl.ds(base, B)], ridx[m], si[m])
        pltpu.async_copy(col_hbm.at[pl.ds(base, B)], cidx[m], si[m])

    def wait_idx(m):
        pltpu.make_async_copy(row_hbm.at[pl.ds(0, B)], ridx[m], si[m]).wait()
        pltpu.make_async_copy(col_hbm.at[pl.ds(0, B)], cidx[m], si[m]).wait()

    def issue_gathers(m3, m2):
        pltpu.async_copy(proj_hbm.at[ridx[m2]], rows[m3], sg[m2])
        pltpu.async_copy(at1_hbm.at[ridx[m2]], a1[m2], sa[m2])
        pltpu.async_copy(at2_hbm.at[cidx[m2]], a2[m2], sa[m2])

    def wait_alpha(m2):
        pltpu.make_async_copy(at1_hbm.at[pl.ds(0, B)], a1[m2], sa[m2]).wait()
        pltpu.make_async_copy(at2_hbm.at[pl.ds(0, B)], a2[m2], sa[m2]).wait()

    def wait_rows(m3, m2):
        pltpu.make_async_copy(proj_hbm.at[pl.ds(0, B)], rows[m3], sg[m2]).wait()

    def compute_ew(m2):
        @plsc.parallel_loop(0, B // 16, 1, unroll=2)
        def _grp(g):
            rv = ridx[m2][pl.ds(g * 16, 16)]
            cv = cidx[m2][pl.ds(g * 16, 16)]
            validf = jnp.where(rv != cv, 1.0, 0.0)
            for j in range(16):
                e = g * 16 + j
                l = a1[m2][e, pl.ds(0, 16)] + a2[m2][e, pl.ds(0, 16)]
                l = jnp.maximum(l, l * NEG)
                w = jnp.exp(l) * validf[j]
                ew[m2][e, pl.ds(0, 16)] = w

    def copy_csc(m2):
        for k in range(B // 16):
            csc[m2][pl.ds(k * 16, 16)] = cidx[m2][pl.ds(k * 16, 16)]

    def scale(m3, m2):
        @plsc.parallel_loop(0, B, 1, unroll=4)
        def _sc(e):
            wv = ew[m2][e, pl.ds(0, 16)]
            for h in range(H):
                wsc = wv[h]
                for v in range(2 * h, 2 * h + 2):
                    rows[m3][e, pl.ds(v * 16, 16)] = (
                        rows[m3][e, pl.ds(v * 16, 16)] * wsc)

    def issue_scatter(m3, m2):
        pltpu.async_copy(rows[m3], acc_sh.at[csc[m2]], ss[m2], add=True)
        pltpu.async_copy(ew[m2], den_sh.at[csc[m2]], ss[m2], add=True)

    def wait_scatter(m3, m2):
        pltpu.make_async_copy(rows[m3], acc_sh.at[csc[m2]], ss[m2]).wait()
        pltpu.make_async_copy(ew[m2], den_sh.at[csc[m2]], ss[m2]).wait()

    NI = NCHUNK // 6

    issue_idx(0, 0)
    wait_idx(0)
    issue_gathers(0, 0)
    issue_idx(1, 1)

    def group(i, carry):
        for k in range(6):
            t = 6 * i + k
            p3, p2 = k % 3, k % 2
            q3, q2 = (k + 1) % 3, (k + 1) % 2

            if k < 2:
                @pl.when(i > 0)
                def _(k=k, q3=q3, p2=p2):
                    wait_scatter(q3, p2)
            else:
                wait_scatter(q3, p2)

            if k == 5:
                @pl.when(i < NI - 1)
                def _(q3=q3, q2=q2):
                    wait_idx(q2)
                    issue_gathers(q3, q2)
            else:
                wait_idx(q2)
                issue_gathers(q3, q2)

            wait_alpha(p2)
            compute_ew(p2)

            wait_rows(p3, p2)
            copy_csc(p2)
            scale(p3, p2)

            if k >= 4:
                @pl.when(i < NI - 1)
                def _(t=t, p2=p2):
                    issue_idx(t + 2, p2)
            else:
                issue_idx(t + 2, p2)

            issue_scatter(p3, p2)
        return carry

    lax.fori_loop(0, NI, group, 0)
    wait_scatter(1, 0)
    wait_scatter(2, 1)

    plsc.subcore_barrier()
    pltpu.async_copy(acc_sh.at[pl.ds(rbase, RPT)], acc_hbm.at[c, pl.ds(rbase, RPT)], sg[0])
    pltpu.async_copy(den_sh.at[pl.ds(rbase, RPT)], den_hbm.at[c, pl.ds(rbase, RPT)], sg[1])
    pltpu.make_async_copy(acc_sh.at[pl.ds(rbase, RPT)], acc_hbm.at[c, pl.ds(rbase, RPT)], sg[0]).wait()
    pltpu.make_async_copy(den_sh.at[pl.ds(rbase, RPT)], den_hbm.at[c, pl.ds(rbase, RPT)], sg[1]).wait()


@jax.jit
def _run(x, edge_index, W, att_src, att_dst, b):
    eye4 = jnp.eye(H, dtype=jnp.float32)
    aw_src = (att_src[:, :, None] * eye4[:, None, :]).reshape(HID, H)
    aw_dst = (att_dst[:, :, None] * eye4[:, None, :]).reshape(HID, H)
    zpad = jnp.zeros((HID, DL - H), jnp.float32)
    aw1 = jnp.concatenate([aw_src, zpad], axis=1)
    aw2 = jnp.concatenate([aw_dst, zpad], axis=1)

    proj, at1, at2 = pl.pallas_call(
        _proj_body,
        grid=(NBLK,),
        in_specs=[
            pl.BlockSpec((N, IN), lambda i: (0, 0)),
            pl.BlockSpec((IN, HID), lambda i: (0, 0)),
            pl.BlockSpec((HID, DL), lambda i: (0, 0)),
            pl.BlockSpec((HID, DL), lambda i: (0, 0)),
        ],
        out_specs=[
            pl.BlockSpec((NPAD, HID), lambda i: (0, 0)),
            pl.BlockSpec((NPAD, DL), lambda i: (0, 0)),
            pl.BlockSpec((NPAD, DL), lambda i: (0, 0)),
        ],
        out_shape=[
            jax.ShapeDtypeStruct((NPAD, HID), jnp.float32),
            jax.ShapeDtypeStruct((NPAD, DL), jnp.float32),
            jax.ShapeDtypeStruct((NPAD, DL), jnp.float32),
        ],
    )(x, W, aw1, aw2)

    pad_ids = N + (jnp.arange(EPAD - E, dtype=jnp.int32) % (NPAD - N))
    row = jnp.concatenate([edge_index[0], pad_ids])
    col = jnp.concatenate([edge_index[1], pad_ids])
    acc, den = _sc_gat(proj, at1, at2, row, col)

    er4 = jnp.repeat(jnp.eye(H, dtype=jnp.float32), C, axis=1)
    er16 = jnp.zeros((DL, HID), jnp.float32).at[:H].set(er4)
    b2 = b.reshape(1, HID)

    out = pl.pallas_call(
        _final_body,
        grid=(NBLK,),
        in_specs=[
            pl.BlockSpec((NC, NPAD, HID), lambda i: (0, 0, 0)),
            pl.BlockSpec((NC, NPAD, DL), lambda i: (0, 0, 0)),
            pl.BlockSpec((NPAD, HID), lambda i: (0, 0)),
            pl.BlockSpec((NPAD, DL), lambda i: (0, 0)),
            pl.BlockSpec((NPAD, DL), lambda i: (0, 0)),
            pl.BlockSpec((H, HID), lambda i: (0, 0)),
            pl.BlockSpec((DL, HID), lambda i: (0, 0)),
            pl.BlockSpec((1, HID), lambda i: (0, 0)),
        ],
        out_specs=pl.BlockSpec((N, HID), lambda i: (0, 0)),
        out_shape=jax.ShapeDtypeStruct((N, HID), jnp.float32),
    )(acc, den, proj, at1, at2, er4, er16, b2)

    return out


def kernel(x, edge_index, W, att_src, att_dst, b):
    return _run(x, edge_index, W, att_src, att_dst, b)


# speedup vs baseline: 1.1823x; 1.0025x over previous
import functools
import jax
import jax.numpy as jnp
from jax import lax
from jax.experimental import pallas as pl
from jax.experimental.pallas import tpu as pltpu
from jax.experimental.pallas import tpu_sc as plsc

N = 10000
E = 320000
IN = 128
H = 4
C = 32
HID = H * C
NEG = 0.2

NPAD = 10240
NC = 2
NS = 16
NW = NC * NS
B = 80
EPW = 10080
EPAD = EPW * NW
NCHUNK = EPW // B
NPAIR = NCHUNK // 2
RPT = NPAD // NS
DL = 16
BLK = 10240
NBLK = NPAD // BLK


def _proj_body(x_ref, w_ref, aw1_ref, aw2_ref, proj_ref, at1_ref, at2_ref):
    p = jnp.dot(x_ref[...], w_ref[...], preferred_element_type=jnp.float32)
    proj_ref[pl.ds(0, N)] = p
    proj_ref[pl.ds(N, NPAD - N)] = jnp.zeros((NPAD - N, HID), jnp.float32)
    at1_ref[pl.ds(0, N)] = jnp.dot(p, aw1_ref[...],
                                   preferred_element_type=jnp.float32)
    at1_ref[pl.ds(N, NPAD - N)] = jnp.zeros((NPAD - N, DL), jnp.float32)
    at2_ref[pl.ds(0, N)] = jnp.dot(p, aw2_ref[...],
                                   preferred_element_type=jnp.float32)
    at2_ref[pl.ds(N, NPAD - N)] = jnp.zeros((NPAD - N, DL), jnp.float32)


def _final_body(acc_ref, den_ref, proj_ref, at1_ref, at2_ref, er4_ref,
                er16_ref, b_ref, o_ref):
    l = at1_ref[pl.ds(0, N), :H] + at2_ref[pl.ds(0, N), :H]
    l = jnp.maximum(l, l * NEG)
    ews = jnp.exp(l)
    ews_e = jnp.dot(ews, er4_ref[...], preferred_element_type=jnp.float32)
    den = den_ref[0, pl.ds(0, N)] + den_ref[1, pl.ds(0, N)]
    den_e = jnp.dot(den, er16_ref[...], preferred_element_type=jnp.float32)
    den_e = den_e + ews_e
    acc = (acc_ref[0, pl.ds(0, N)] + acc_ref[1, pl.ds(0, N)]
           + ews_e * proj_ref[pl.ds(0, N)])
    o_ref[...] = acc / den_e + b_ref[...]


_sc_mesh = plsc.VectorSubcoreMesh(core_axis_name="c", subcore_axis_name="s")


@functools.partial(
    pl.kernel,
    out_type=(
        jax.ShapeDtypeStruct((NC, NPAD, HID), jnp.float32),
        jax.ShapeDtypeStruct((NC, NPAD, DL), jnp.float32),
    ),
    mesh=_sc_mesh,
    compiler_params=pltpu.CompilerParams(use_tc_tiling_on_sc=False,
                                         needs_layout_passes=False),
    scratch_types=[
        [pltpu.VMEM((B, HID), jnp.float32) for _ in range(3)],
        [pltpu.VMEM((B, DL), jnp.float32) for _ in range(2)],
        [pltpu.VMEM((B, DL), jnp.float32) for _ in range(2)],
        [pltpu.VMEM((B,), jnp.int32) for _ in range(2)],
        [pltpu.VMEM((B,), jnp.int32) for _ in range(2)],
        [pltpu.VMEM((B,), jnp.int32) for _ in range(2)],
        [pltpu.VMEM((B, DL), jnp.float32) for _ in range(2)],
        pltpu.VMEM_SHARED((NPAD, HID), jnp.float32),
        pltpu.VMEM_SHARED((NPAD, DL), jnp.float32),
        [pltpu.SemaphoreType.DMA for _ in range(2)],
        [pltpu.SemaphoreType.DMA for _ in range(2)],
        [pltpu.SemaphoreType.DMA for _ in range(2)],
        [pltpu.SemaphoreType.DMA for _ in range(2)],
    ],
)
def _sc_gat(proj_hbm, at1_hbm, at2_hbm, row_hbm, col_hbm, acc_hbm, den_hbm,
            rows, a1, a2, ridx, cidx, csc, ew, acc_sh, den_sh,
            sg, sa, si, ss):
    c = lax.axis_index("c")
    s = lax.axis_index("s")
    wid = c * NS + s
    ebase = wid * EPW

    zf = jnp.zeros((16,), jnp.float32)

    def _zrow(i, carry):
        for v in range(HID // 16):
            rows[0][i, pl.ds(v * 16, 16)] = zf
        ew[0][i, pl.ds(0, 16)] = zf
        return carry

    lax.fori_loop(0, B, _zrow, 0)

    rbase = s * RPT
    for k in range(RPT // B):
        pltpu.async_copy(rows[0], acc_sh.at[pl.ds(rbase + k * B, B)], sg[0])
        pltpu.async_copy(ew[0], den_sh.at[pl.ds(rbase + k * B, B)], sg[0])
    for k in range(RPT // B):
        pltpu.make_async_copy(rows[0], acc_sh.at[pl.ds(rbase + k * B, B)], sg[0]).wait()
        pltpu.make_async_copy(ew[0], den_sh.at[pl.ds(rbase + k * B, B)], sg[0]).wait()
    plsc.subcore_barrier()

    def issue_idx(t, m):
        base = ebase + t * B
        pltpu.async_copy(row_hbm.at[pl.ds(base, B)], ridx[m], si[m])
        pltpu.async_copy(col_hbm.at[pl.ds(base, B)], cidx[m], si[m])

    def wait_idx(m):
        pltpu.make_async_copy(row_hbm.at[pl.ds(0, B)], ridx[m], si[m]).wait()
        pltpu.make_async_copy(col_hbm.at[pl.ds(0, B)], cidx[m], si[m]).wait()

    def issue_gathers(m3, m2):
        pltpu.async_copy(proj_hbm.at[ridx[m2]], rows[m3], sg[m2])
        pltpu.async_copy(at1_hbm.at[ridx[m2]], a1[m2], sa[m2])
        pltpu.async_copy(at2_hbm.at[cidx[m2]], a2[m2], sa[m2])

    def wait_alpha(m2):
        pltpu.make_async_copy(at1_hbm.at[pl.ds(0, B)], a1[m2], sa[m2]).wait()
        pltpu.make_async_copy(at2_hbm.at[pl.ds(0, B)], a2[m2], sa[m2]).wait()

    def wait_rows(m3, m2):
        pltpu.make_async_copy(proj_hbm.at[pl.ds(0, B)], rows[m3], sg[m2]).wait()

    def compute_ew(m2):
        @plsc.parallel_loop(0, B // 16, 1, unroll=2)
        def _grp(g):
            rv = ridx[m2][pl.ds(g * 16, 16)]
            cv = cidx[m2][pl.ds(g * 16, 16)]
            validf = jnp.where(rv != cv, 1.0, 0.0)
            for j in range(16):
                e = g * 16 + j
                l = a1[m2][e, pl.ds(0, 16)] + a2[m2][e, pl.ds(0, 16)]
                l = jnp.maximum(l, l * NEG)
                w = jnp.exp(l) * validf[j]
                ew[m2][e, pl.ds(0, 16)] = w

    def copy_csc(m2):
        for k in range(B // 16):
            csc[m2][pl.ds(k * 16, 16)] = cidx[m2][pl.ds(k * 16, 16)]

    hsplat = [jnp.full((16,), h, jnp.int32) for h in range(H)]

    def scale(m3, m2):
        @plsc.parallel_loop(0, B, 1, unroll=4)
        def _sc(e):
            wv = ew[m2][e, pl.ds(0, 16)]
            for h in range(H):
                wb = wv[hsplat[h]]
                for v in range(2 * h, 2 * h + 2):
                    rows[m3][e, pl.ds(v * 16, 16)] = (
                        rows[m3][e, pl.ds(v * 16, 16)] * wb)

    def issue_scatter(m3, m2):
        pltpu.async_copy(rows[m3], acc_sh.at[csc[m2]], ss[m2], add=True)
        pltpu.async_copy(ew[m2], den_sh.at[csc[m2]], ss[m2], add=True)

    def wait_scatter(m3, m2):
        pltpu.make_async_copy(rows[m3], acc_sh.at[csc[m2]], ss[m2]).wait()
        pltpu.make_async_copy(ew[m2], den_sh.at[csc[m2]], ss[m2]).wait()

    NI = NCHUNK // 6

    issue_idx(0, 0)
    wait_idx(0)
    issue_gathers(0, 0)
    issue_idx(1, 1)

    def group(i, carry):
        for k in range(6):
            t = 6 * i + k
            p3, p2 = k % 3, k % 2
            q3, q2 = (k + 1) % 3, (k + 1) % 2

            if k < 2:
                @pl.when(i > 0)
                def _(k=k, q3=q3, p2=p2):
                    wait_scatter(q3, p2)
            else:
                wait_scatter(q3, p2)

            if k == 5:
                @pl.when(i < NI - 1)
                def _(q3=q3, q2=q2):
                    wait_idx(q2)
                    issue_gathers(q3, q2)
            else:
                wait_idx(q2)
                issue_gathers(q3, q2)

            wait_alpha(p2)
            compute_ew(p2)

            wait_rows(p3, p2)
            copy_csc(p2)
            scale(p3, p2)

            if k >= 4:
                @pl.when(i < NI - 1)
                def _(t=t, p2=p2):
                    issue_idx(t + 2, p2)
            else:
                issue_idx(t + 2, p2)

            issue_scatter(p3, p2)
        return carry

    lax.fori_loop(0, NI, group, 0)
    wait_scatter(1, 0)
    wait_scatter(2, 1)

    plsc.subcore_barrier()
    pltpu.async_copy(acc_sh.at[pl.ds(rbase, RPT)], acc_hbm.at[c, pl.ds(rbase, RPT)], sg[0])
    pltpu.async_copy(den_sh.at[pl.ds(rbase, RPT)], den_hbm.at[c, pl.ds(rbase, RPT)], sg[1])
    pltpu.make_async_copy(acc_sh.at[pl.ds(rbase, RPT)], acc_hbm.at[c, pl.ds(rbase, RPT)], sg[0]).wait()
    pltpu.make_async_copy(den_sh.at[pl.ds(rbase, RPT)], den_hbm.at[c, pl.ds(rbase, RPT)], sg[1]).wait()


@jax.jit
def _run(x, edge_index, W, att_src, att_dst, b):
    eye4 = jnp.eye(H, dtype=jnp.float32)
    aw_src = (att_src[:, :, None] * eye4[:, None, :]).reshape(HID, H)
    aw_dst = (att_dst[:, :, None] * eye4[:, None, :]).reshape(HID, H)
    zpad = jnp.zeros((HID, DL - H), jnp.float32)
    aw1 = jnp.concatenate([aw_src, zpad], axis=1)
    aw2 = jnp.concatenate([aw_dst, zpad], axis=1)

    proj, at1, at2 = pl.pallas_call(
        _proj_body,
        grid=(NBLK,),
        in_specs=[
            pl.BlockSpec((N, IN), lambda i: (0, 0)),
            pl.BlockSpec((IN, HID), lambda i: (0, 0)),
            pl.BlockSpec((HID, DL), lambda i: (0, 0)),
            pl.BlockSpec((HID, DL), lambda i: (0, 0)),
        ],
        out_specs=[
            pl.BlockSpec((NPAD, HID), lambda i: (0, 0)),
            pl.BlockSpec((NPAD, DL), lambda i: (0, 0)),
            pl.BlockSpec((NPAD, DL), lambda i: (0, 0)),
        ],
        out_shape=[
            jax.ShapeDtypeStruct((NPAD, HID), jnp.float32),
            jax.ShapeDtypeStruct((NPAD, DL), jnp.float32),
            jax.ShapeDtypeStruct((NPAD, DL), jnp.float32),
        ],
    )(x, W, aw1, aw2)

    pad_ids = N + (jnp.arange(EPAD - E, dtype=jnp.int32) % (NPAD - N))
    row = jnp.concatenate([edge_index[0], pad_ids])
    col = jnp.concatenate([edge_index[1], pad_ids])
    acc, den = _sc_gat(proj, at1, at2, row, col)

    er4 = jnp.repeat(jnp.eye(H, dtype=jnp.float32), C, axis=1)
    er16 = jnp.zeros((DL, HID), jnp.float32).at[:H].set(er4)
    b2 = b.reshape(1, HID)

    out = pl.pallas_call(
        _final_body,
        grid=(NBLK,),
        in_specs=[
            pl.BlockSpec((NC, NPAD, HID), lambda i: (0, 0, 0)),
            pl.BlockSpec((NC, NPAD, DL), lambda i: (0, 0, 0)),
            pl.BlockSpec((NPAD, HID), lambda i: (0, 0)),
            pl.BlockSpec((NPAD, DL), lambda i: (0, 0)),
            pl.BlockSpec((NPAD, DL), lambda i: (0, 0)),
            pl.BlockSpec((H, HID), lambda i: (0, 0)),
            pl.BlockSpec((DL, HID), lambda i: (0, 0)),
            pl.BlockSpec((1, HID), lambda i: (0, 0)),
        ],
        out_specs=pl.BlockSpec((N, HID), lambda i: (0, 0)),
        out_shape=jax.ShapeDtypeStruct((N, HID), jnp.float32),
    )(acc, den, proj, at1, at2, er4, er16, b2)

    return out


def kernel(x, edge_index, W, att_src, att_dst, b):
    return _run(x, edge_index, W, att_src, att_dst, b)
